# baseline (speedup 1.0000x reference)
import functools

import jax
import jax.numpy as jnp
from jax import lax
from jax.experimental import pallas as pl
from jax.experimental.pallas import tpu as pltpu
from jax.experimental.pallas import tpu_sc as plsc

VOCAB = 21128
DIM = 768
BATCH = 4096
SEQ = 50
TAIL = SEQ % 8
MAIN = SEQ - TAIL

_info = plsc.get_sparse_core_info()
NC, NS = _info.num_cores, _info.num_subcores
NW = NC * NS
BATCH_PER_W = BATCH // NW
MAIN_PER_W = BATCH_PER_W * MAIN
TAIL_PER_W = BATCH_PER_W * TAIL
GROUP = 8
N_GROUPS = BATCH_PER_W // GROUP
NBUF = 2


def _make_kernel_real():
    mesh = plsc.VectorSubcoreMesh(core_axis_name="c", subcore_axis_name="s")

    @functools.partial(
        pl.kernel,
        out_type=jax.ShapeDtypeStruct((BATCH, SEQ, DIM), jnp.float32),
        mesh=mesh,
        scratch_types=[
            pltpu.VMEM((MAIN_PER_W,), jnp.int32),
            pltpu.VMEM((TAIL_PER_W,), jnp.int32),
            pltpu.VMEM((NBUF, SEQ, DIM), jnp.float32),
            pltpu.VMEM((NBUF, GROUP * TAIL, DIM), jnp.float32),
            [pltpu.SemaphoreType.DMA] * NBUF,
            [pltpu.SemaphoreType.DMA] * NBUF,
            [pltpu.SemaphoreType.DMA] * NBUF,
        ],
    )
    def k(midx_hbm, tidx_hbm, table_hbm, out_hbm, midx_v, tidx_v, rows_v,
          tails_v, gsems, ssems, tsems):
        wid = lax.axis_index("s") * NC + lax.axis_index("c")
        bbase = wid * BATCH_PER_W
        pltpu.sync_copy(midx_hbm.at[pl.ds(wid * MAIN_PER_W, MAIN_PER_W)], midx_v)
        pltpu.sync_copy(tidx_hbm.at[pl.ds(wid * TAIL_PER_W, TAIL_PER_W)], tidx_v)

        def gather(j, slot):
            pltpu.async_copy(
                table_hbm.at[midx_v.at[pl.ds(j * MAIN, MAIN)]],
                rows_v.at[slot].at[pl.ds(0, MAIN)],
                gsems[slot],
            )

        def wait_gather(j, slot):
            pltpu.make_async_copy(
                table_hbm.at[midx_v.at[pl.ds(j * MAIN, MAIN)]],
                rows_v.at[slot].at[pl.ds(0, MAIN)],
                gsems[slot],
            ).wait()

        def tgather(g, tslot):
            pltpu.async_copy(
                table_hbm.at[tidx_v.at[pl.ds(g * GROUP * TAIL, GROUP * TAIL)]],
                tails_v.at[tslot],
                tsems[tslot],
            )

        def wait_tgather(g, tslot):
            pltpu.make_async_copy(
                table_hbm.at[tidx_v.at[pl.ds(g * GROUP * TAIL, GROUP * TAIL)]],
                tails_v.at[tslot],
                tsems[tslot],
            ).wait()

        def store(j, slot):
            pltpu.async_copy(rows_v.at[slot], out_hbm.at[bbase + j], ssems[slot])

        def wait_store(j, slot):
            pltpu.make_async_copy(
                rows_v.at[slot], out_hbm.at[bbase + j], ssems[slot]
            ).wait()

        gather(0, 0)
        tgather(0, 0)
        tgather(1, 1)

        SUPER = 2 * GROUP

        def body(gg, _):
            for t in range(SUPER):
                j = gg * SUPER + t
                b = t % NBUF
                ns = (b + 1) % NBUF
                tslot = (t // GROUP) % 2
                rg = t % GROUP
                g = gg * 2 + t // GROUP

                if t % GROUP == 0:
                    wait_tgather(g, tslot)

                @pl.when(j >= 1)
                def _():
                    wait_store(j - 1, ns)

                @pl.when(j + 1 < BATCH_PER_W)
                def _():
                    gather(j + 1, ns)

                wait_gather(j, b)
                for r in range(TAIL):
                    for c in range(DIM // 16):
                        rows_v[b, MAIN + r, pl.ds(c * 16, 16)] = tails_v[
                            tslot, rg * TAIL + r, pl.ds(c * 16, 16)
                        ]
                store(j, b)

                if t % GROUP == GROUP - 1:
                    @pl.when(g + 2 < N_GROUPS)
                    def _():
                        tgather(g + 2, tslot)
            return 0

        lax.fori_loop(0, BATCH_PER_W // SUPER, body, 0, unroll=False)
        wait_store(BATCH_PER_W - 1, (BATCH_PER_W - 1) % NBUF)

    return k


_gather_fn = _make_kernel_real()


def kernel(token_ids, table):
    ids = token_ids.astype(jnp.int32)
    main_ids = ids[:, :MAIN].reshape(NW, MAIN_PER_W).reshape(NW * MAIN_PER_W)
    tail_ids = ids[:, MAIN:].reshape(NW, TAIL_PER_W).reshape(NW * TAIL_PER_W)
    return _gather_fn(main_ids, tail_ids, table)


# speedup vs baseline: 1.0382x; 1.0382x over previous
import functools

import jax
import jax.numpy as jnp
from jax import lax
from jax.experimental import pallas as pl
from jax.experimental.pallas import tpu as pltpu
from jax.experimental.pallas import tpu_sc as plsc

VOCAB = 21128
DIM = 768
BATCH = 4096
SEQ = 50
TAIL = SEQ % 8
MAIN = SEQ - TAIL

_info = plsc.get_sparse_core_info()
NC, NS = _info.num_cores, _info.num_subcores
NW = NC * NS
BATCH_PER_W = BATCH // NW
MAIN_PER_W = BATCH_PER_W * MAIN
TAIL_PER_W = BATCH_PER_W * TAIL
GROUP = 8
N_GROUPS = BATCH_PER_W // GROUP
NBUF = 2


def _make_kernel():
    mesh = plsc.VectorSubcoreMesh(core_axis_name="c", subcore_axis_name="s")

    @functools.partial(
        pl.kernel,
        out_type=jax.ShapeDtypeStruct((BATCH, SEQ, DIM), jnp.float32),
        mesh=mesh,
        scratch_types=[
            pltpu.VMEM((MAIN_PER_W,), jnp.int32),
            pltpu.VMEM((TAIL_PER_W,), jnp.int32),
            pltpu.VMEM((NBUF, MAIN, DIM), jnp.float32),
            pltpu.VMEM((GROUP * TAIL, DIM), jnp.float32),
            pltpu.VMEM((2, GROUP, TAIL, DIM), jnp.float32),
            [pltpu.SemaphoreType.DMA] * NBUF,
            [pltpu.SemaphoreType.DMA] * NBUF,
            pltpu.SemaphoreType.DMA,
            [pltpu.SemaphoreType.DMA] * 2,
        ],
    )
    def k(midx_hbm, tidx_hbm, table_hbm, out_hbm, midx_v, tidx_v, rows_v,
          tails_v, tstore_v, gsems, ssems, tgsem, tssems):
        wid = lax.axis_index("s") * NC + lax.axis_index("c")
        bbase = wid * BATCH_PER_W
        pltpu.sync_copy(midx_hbm.at[pl.ds(wid * MAIN_PER_W, MAIN_PER_W)], midx_v)
        pltpu.sync_copy(tidx_hbm.at[pl.ds(wid * TAIL_PER_W, TAIL_PER_W)], tidx_v)

        def gather(j, slot):
            pltpu.async_copy(
                table_hbm.at[midx_v.at[pl.ds(j * MAIN, MAIN)]],
                rows_v.at[slot],
                gsems[slot],
            )

        def wait_gather(j, slot):
            pltpu.make_async_copy(
                table_hbm.at[midx_v.at[pl.ds(j * MAIN, MAIN)]],
                rows_v.at[slot],
                gsems[slot],
            ).wait()

        def tgather(g):
            pltpu.async_copy(
                table_hbm.at[tidx_v.at[pl.ds(g * GROUP * TAIL, GROUP * TAIL)]],
                tails_v,
                tgsem,
            )

        def wait_tgather(g):
            pltpu.make_async_copy(
                table_hbm.at[tidx_v.at[pl.ds(g * GROUP * TAIL, GROUP * TAIL)]],
                tails_v,
                tgsem,
            ).wait()

        def store(j, slot):
            pltpu.async_copy(
                rows_v.at[slot],
                out_hbm.at[bbase + j, pl.ds(0, MAIN)],
                ssems[slot],
            )

        def wait_store(j, slot):
            pltpu.make_async_copy(
                rows_v.at[slot],
                out_hbm.at[bbase + j, pl.ds(0, MAIN)],
                ssems[slot],
            ).wait()

        def tstore(g, tslot):
            pltpu.async_copy(
                tstore_v.at[tslot],
                out_hbm.at[pl.ds(bbase + g * GROUP, GROUP), pl.ds(MAIN, TAIL)],
                tssems[tslot],
            )

        def wait_tstore(g, tslot):
            pltpu.make_async_copy(
                tstore_v.at[tslot],
                out_hbm.at[pl.ds(bbase + g * GROUP, GROUP), pl.ds(MAIN, TAIL)],
                tssems[tslot],
            ).wait()

        gather(0, 0)
        tgather(0)

        SUPER = 2 * GROUP

        def body(gg, _):
            for t in range(SUPER):
                j = gg * SUPER + t
                b = t % NBUF
                ns = (b + 1) % NBUF
                rg = t % GROUP
                tslot = (t // GROUP) % 2
                g = gg * 2 + t // GROUP

                if rg == 0:
                    wait_tgather(g)

                    @pl.when(g >= 2)
                    def _():
                        wait_tstore(g - 2, tslot)

                for r in range(TAIL):
                    for c in range(DIM // 16):
                        tstore_v[tslot, rg, r, pl.ds(c * 16, 16)] = tails_v[
                            rg * TAIL + r, pl.ds(c * 16, 16)
                        ]

                if rg == GROUP - 1:
                    tstore(g, tslot)

                    @pl.when(g + 1 < N_GROUPS)
                    def _():
                        tgather(g + 1)

                @pl.when(j >= 1)
                def _():
                    wait_store(j - 1, ns)

                @pl.when(j + 1 < BATCH_PER_W)
                def _():
                    gather(j + 1, ns)

                wait_gather(j, b)
                store(j, b)
            return 0

        lax.fori_loop(0, BATCH_PER_W // SUPER, body, 0, unroll=False)
        wait_store(BATCH_PER_W - 1, (BATCH_PER_W - 1) % NBUF)
        wait_tstore(N_GROUPS - 2, 0)
        wait_tstore(N_GROUPS - 1, 1)

    return k


_gather_fn = _make_kernel()


def kernel(token_ids, table):
    ids = token_ids.astype(jnp.int32)
    main_ids = ids[:, :MAIN].reshape(NW * MAIN_PER_W)
    tail_ids = ids[:, MAIN:].reshape(NW * TAIL_PER_W)
    return _gather_fn(main_ids, tail_ids, table)
